# Initial kernel scaffold; baseline (speedup 1.0000x reference)
import jax
import jax.numpy as jnp
from jax.experimental import pallas as pl


def kernel(x, patch_W, patch_b, cls_token, pos_embed, router_W, router_b, ln1_g, ln1_b, Wv, bv, Wo, bo, ln2_g, ln2_b, W1, b1, W2, b2, norm_g, norm_b, head_W, head_b):
    raise NotImplementedError("write your pallas kernel here")


# speedup vs baseline: 46.2407x; 46.2407x over previous
import jax
import jax.numpy as jnp
from jax.experimental import pallas as pl
from jax.experimental.pallas import tpu as pltpu

EMB = 384
NEXP = 8
HID = 1536
NCLS = 1000
TOPK = 2
_EPS = 1e-5


def _layernorm(v, g, b):
    mu = jnp.mean(v, axis=-1, keepdims=True)
    var = jnp.mean((v - mu) ** 2, axis=-1, keepdims=True)
    return (v - mu) / jnp.sqrt(var + _EPS) * g + b


def _mm_t(a, w):
    return jax.lax.dot_general(
        a, w, (((1,), (1,)), ((), ())), preferred_element_type=jnp.float32
    )


def _router_body(cls_ref, pos_ref, w_ref, b_ref, idx_ref):
    tokrow = cls_ref[...] + pos_ref[...]
    logits = _mm_t(tokrow, w_ref[...]) + b_ref[...]
    lane = jax.lax.broadcasted_iota(jnp.int32, logits.shape, 1)
    m1 = jnp.max(logits, axis=-1, keepdims=True)
    i1 = jnp.min(jnp.where(logits == m1, lane, NEXP), axis=-1, keepdims=True)
    masked = jnp.where(lane == i1, jnp.full_like(logits, -3.0e38), logits)
    m2 = jnp.max(masked, axis=-1, keepdims=True)
    i2 = jnp.min(jnp.where(masked == m2, lane, NEXP), axis=-1, keepdims=True)
    idx_ref[...] = jnp.concatenate([i1, i2], axis=-1)


def _moe_body(eidx_ref, cls_ref, pos_ref, g1_ref, c1_ref, wv_ref, bv_ref,
              wo_ref, bo_ref, g2_ref, c2_ref, w1_ref, b1_ref, w2_ref, b2_ref,
              ng_ref, nb_ref, hw_ref, hb_ref, out_ref, acc_ref):
    i = pl.program_id(0)
    tokrow = cls_ref[...] + pos_ref[...]
    xn = _layernorm(tokrow, g1_ref[0], c1_ref[0])
    v = _mm_t(xn, wv_ref[0]) + bv_ref[0]
    attn = _mm_t(v, wo_ref[0]) + bo_ref[0]
    hmid = tokrow + attn
    hn = _layernorm(hmid, g2_ref[0], c2_ref[0])
    h0 = _mm_t(hn, w1_ref[0]) + b1_ref[0]
    h1 = 0.5 * h0 * (1.0 + jax.lax.erf(h0 * (1.0 / jnp.sqrt(2.0).astype(jnp.float32))))
    m = _mm_t(h1, w2_ref[0]) + b2_ref[0]
    y = hmid + m

    @pl.when(i == 0)
    def _():
        acc_ref[...] = y

    @pl.when(i == 1)
    def _():
        s = (acc_ref[...] + y) * (1.0 / TOPK)
        o = _layernorm(s, ng_ref[...], nb_ref[...])
        logits = _mm_t(o, hw_ref[...]) + hb_ref[...]
        out_ref[...] = jnp.broadcast_to(logits, out_ref.shape)


def kernel(x, patch_W, patch_b, cls_token, pos_embed, router_W, router_b,
           ln1_g, ln1_b, Wv, bv, Wo, bo, ln2_g, ln2_b, W1, b1, W2, b2,
           norm_g, norm_b, head_W, head_b):
    Bsz = x.shape[0]
    cls2 = cls_token.reshape(1, EMB)
    pos0 = pos_embed[:, 0, :].reshape(1, EMB)

    idx = pl.pallas_call(
        _router_body,
        out_shape=jax.ShapeDtypeStruct((1, TOPK), jnp.int32),
    )(cls2, pos0, router_W, router_b.reshape(1, NEXP))
    eidx = idx.reshape(TOPK)

    g1r = ln1_g.reshape(NEXP, 1, EMB)
    c1r = ln1_b.reshape(NEXP, 1, EMB)
    bvr = bv.reshape(NEXP, 1, EMB)
    bor = bo.reshape(NEXP, 1, EMB)
    g2r = ln2_g.reshape(NEXP, 1, EMB)
    c2r = ln2_b.reshape(NEXP, 1, EMB)
    b1r = b1.reshape(NEXP, 1, HID)
    b2r = b2.reshape(NEXP, 1, EMB)

    def _vec(d):
        return pl.BlockSpec((1, 1, d), lambda i, e: (e[i], 0, 0))

    def _mat(r, c):
        return pl.BlockSpec((1, r, c), lambda i, e: (e[i], 0, 0))

    def _full(shape):
        nd = len(shape)
        return pl.BlockSpec(shape, lambda i, e: (0,) * nd)

    grid_spec = pltpu.PrefetchScalarGridSpec(
        num_scalar_prefetch=1,
        grid=(TOPK,),
        in_specs=[
            _full((1, EMB)),
            _full((1, EMB)),
            _vec(EMB),
            _vec(EMB),
            _mat(EMB, EMB),
            _vec(EMB),
            _mat(EMB, EMB),
            _vec(EMB),
            _vec(EMB),
            _vec(EMB),
            _mat(HID, EMB),
            _vec(HID),
            _mat(EMB, HID),
            _vec(EMB),
            _full((1, EMB)),
            _full((1, EMB)),
            _full((NCLS, EMB)),
            _full((1, NCLS)),
        ],
        out_specs=pl.BlockSpec((Bsz, NCLS), lambda i, e: (0, 0)),
        scratch_shapes=[pltpu.VMEM((1, EMB), jnp.float32)],
    )

    out = pl.pallas_call(
        _moe_body,
        grid_spec=grid_spec,
        out_shape=jax.ShapeDtypeStruct((Bsz, NCLS), jnp.float32),
    )(eidx, cls2, pos0, g1r, c1r, Wv, bvr, Wo, bor, g2r, c2r,
      W1, b1r, W2, b2r, norm_g.reshape(1, EMB), norm_b.reshape(1, EMB),
      head_W, head_b.reshape(1, NCLS))
    return out
